# baseline (speedup 1.0000x reference)
import functools

import jax
import jax.numpy as jnp
import numpy as np
from jax.experimental import pallas as pl
from jax.experimental.pallas import tpu as pltpu

TOKENS = 8192
HIDDEN = 2048
NUM_EXPERTS = 64
TOP_K = 8
AUX_COEF = 0.01
Z_COEF = 0.001

BLOCK_T = 1024
NCHUNKS = 4
CT = BLOCK_T // NCHUNKS


@functools.lru_cache(maxsize=1)
def _noise_np():
    with jax.ensure_compile_time_eval():
        return np.asarray(
            jax.random.normal(jax.random.key(42), (TOKENS, NUM_EXPERTS), dtype=jnp.float32)
        )


def _noise_const():
    try:
        return _noise_np()
    except Exception:
        return jax.random.normal(jax.random.key(42), (TOKENS, NUM_EXPERTS), dtype=jnp.float32)


def _route_chunk(logits, clean, gates_ref, row0):
    running = logits
    for j in range(TOP_K):
        m = jnp.max(running, axis=-1, keepdims=True)
        if j == 0:
            rowmax = m
        running = jnp.where(running == m, -jnp.inf, running)
        thresh = m

    sel = logits >= thresh
    e = jnp.exp(logits - rowmax)
    sel_e = jnp.where(sel, e, 0.0)
    denom = jnp.sum(sel_e, axis=-1, keepdims=True)
    gates = sel_e / denom
    gates_ref[pl.ds(row0, CT), :] = gates

    imp_partial = jnp.sum(gates, axis=0, keepdims=True)
    loads_partial = jnp.sum((gates > 0.0).astype(jnp.float32), axis=0, keepdims=True)
    stats = jnp.concatenate([imp_partial, loads_partial], axis=1)

    cmax = jnp.max(clean, axis=-1, keepdims=True)
    lse = jnp.log(jnp.sum(jnp.exp(clean - cmax), axis=-1, keepdims=True)) + cmax
    z_partial = jnp.reshape(jnp.sum(lse * lse), (1, 1))
    return stats, z_partial


def _noisy(logits_all, noise_ref, row0):
    clean = logits_all[:, :NUM_EXPERTS]
    raw_noise = logits_all[:, NUM_EXPERTS:]
    stddev = jax.nn.softplus(raw_noise) + 1e-10
    logits = clean + noise_ref[pl.ds(row0, CT), :] * stddev
    return logits, clean


def _router_kernel(x_ref, wg_ref, wn_ref, noise_ref, gates_ref, loss_ref,
                   acc_ref, wt_ref, lp_ref, nblocks):
    i = pl.program_id(0)

    @pl.when(i == 0)
    def _():
        wt_ref[:, :NUM_EXPERTS] = wg_ref[...].T
        wt_ref[:, NUM_EXPERTS:] = wn_ref[...].T
        lp_ref[...] = jnp.zeros_like(lp_ref)
        acc_ref[...] = jnp.zeros_like(acc_ref)

    w = wt_ref[...]
    base = i * BLOCK_T

    stats = jnp.zeros((1, 2 * NUM_EXPERTS), jnp.float32)
    z_partial = jnp.zeros((1, 1), jnp.float32)

    prev0 = jnp.maximum(base - CT, 0)
    logits_p, clean_p = _noisy(lp_ref[...], noise_ref, prev0)
    sp, zp = _route_chunk(logits_p, clean_p, gates_ref, prev0)
    scale = jnp.where(i > 0, 1.0, 0.0)
    stats = stats + sp * scale
    z_partial = z_partial + zp * scale

    chunks = []
    for c in range(NCHUNKS):
        r0 = base + c * CT
        logits_all = jnp.dot(x_ref[pl.ds(c * CT, CT), :], w,
                             preferred_element_type=jnp.float32)
        if c == NCHUNKS - 1:
            lp_ref[...] = logits_all
        else:
            chunks.append((logits_all, r0))

    for logits_all, r0 in chunks:
        logits, clean = _noisy(logits_all, noise_ref, r0)
        s, z = _route_chunk(logits, clean, gates_ref, r0)
        stats = stats + s
        z_partial = z_partial + z

    acc_ref[0:1, :] += stats
    acc_ref[1:2, 0:1] += z_partial

    @pl.when(i == nblocks - 1)
    def _():
        last0 = base + (NCHUNKS - 1) * CT
        logits_l, clean_l = _noisy(lp_ref[...], noise_ref, last0)
        sl, zl_p = _route_chunk(logits_l, clean_l, gates_ref, last0)
        imp = acc_ref[0:1, :NUM_EXPERTS] + sl[0:1, :NUM_EXPERTS]
        loads = acc_ref[0:1, NUM_EXPERTS:] + sl[0:1, NUM_EXPERTS:]
        zsum = acc_ref[1, 0] + zl_p[0, 0]
        lb = AUX_COEF * (NUM_EXPERTS * jnp.sum(imp * loads) / float(TOKENS * TOKENS))
        zl = Z_COEF * zsum / float(TOKENS)
        loss_ref[...] = jnp.reshape(lb + zl, (1, 1))


def kernel(x, w_gate, w_noise):
    noise = jnp.asarray(_noise_const())
    nblocks = TOKENS // BLOCK_T

    gates, loss = pl.pallas_call(
        functools.partial(_router_kernel, nblocks=nblocks),
        grid=(nblocks,),
        in_specs=[
            pl.BlockSpec((BLOCK_T, HIDDEN), lambda i: (i, 0)),
            pl.BlockSpec((NUM_EXPERTS, HIDDEN), lambda i: (0, 0)),
            pl.BlockSpec((NUM_EXPERTS, HIDDEN), lambda i: (0, 0)),
            pl.BlockSpec((TOKENS, NUM_EXPERTS), lambda i: (0, 0)),
        ],
        out_specs=[
            pl.BlockSpec((TOKENS, NUM_EXPERTS), lambda i: (0, 0)),
            pl.BlockSpec((1, 1), lambda i: (0, 0)),
        ],
        out_shape=[
            jax.ShapeDtypeStruct((TOKENS, NUM_EXPERTS), jnp.float32),
            jax.ShapeDtypeStruct((1, 1), jnp.float32),
        ],
        scratch_shapes=[
            pltpu.VMEM((8, 2 * NUM_EXPERTS), jnp.float32),
            pltpu.VMEM((HIDDEN, 2 * NUM_EXPERTS), jnp.float32),
            pltpu.VMEM((CT, 2 * NUM_EXPERTS), jnp.float32),
        ],
        compiler_params=pltpu.CompilerParams(
            dimension_semantics=("arbitrary",),
        ),
    )(x, w_gate, w_noise, noise)
    return gates, jnp.reshape(loss, ())


# speedup vs baseline: 1.0600x; 1.0600x over previous
import functools

import jax
import jax.numpy as jnp
import numpy as np
from jax.experimental import pallas as pl
from jax.experimental.pallas import tpu as pltpu

TOKENS = 8192
HIDDEN = 2048
NUM_EXPERTS = 64
TOP_K = 8
AUX_COEF = 0.01
Z_COEF = 0.001

BLOCK_T = 1024


@functools.lru_cache(maxsize=1)
def _noise_np():
    with jax.ensure_compile_time_eval():
        return np.asarray(
            jax.random.normal(jax.random.key(42), (TOKENS, NUM_EXPERTS), dtype=jnp.float32)
        )


def _noise_const():
    try:
        return _noise_np()
    except Exception:
        return jax.random.normal(jax.random.key(42), (TOKENS, NUM_EXPERTS), dtype=jnp.float32)


NCHUNKS = 4


def _route_chunk(logits, clean, gates_ref, row0, bt):
    running = logits
    for j in range(TOP_K):
        m = jnp.max(running, axis=-1, keepdims=True)
        if j == 0:
            rowmax = m
        running = jnp.where(running == m, -jnp.inf, running)
        thresh = m

    sel = logits >= thresh
    e = jnp.exp(logits - rowmax)
    sel_e = jnp.where(sel, e, 0.0)
    denom = jnp.sum(sel_e, axis=-1, keepdims=True)
    gates = sel_e / denom
    gates_ref[pl.ds(row0, bt), :] = gates

    imp_partial = jnp.sum(gates, axis=0, keepdims=True)
    loads_partial = jnp.sum((gates > 0.0).astype(jnp.float32), axis=0, keepdims=True)
    stats = jnp.concatenate([imp_partial, loads_partial], axis=1)

    cmax = jnp.max(clean, axis=-1, keepdims=True)
    lse = jnp.log(jnp.sum(jnp.exp(clean - cmax), axis=-1, keepdims=True)) + cmax
    z_partial = jnp.reshape(jnp.sum(lse * lse), (1, 1))
    return stats, z_partial


def _router_kernel(x_ref, wg_ref, wn_ref, noise_ref, gates_ref, loss_ref,
                   acc_ref, wt_ref, nblocks):
    i = pl.program_id(0)

    @pl.when(i == 0)
    def _():
        wt_ref[:, :NUM_EXPERTS] = wg_ref[...].T
        wt_ref[:, NUM_EXPERTS:] = wn_ref[...].T

    w = wt_ref[...]
    ct = BLOCK_T // NCHUNKS

    chunks = []
    for c in range(NCHUNKS):
        r0 = c * ct
        logits_all = jnp.dot(x_ref[pl.ds(r0, ct), :], w,
                             preferred_element_type=jnp.float32)
        clean = logits_all[:, :NUM_EXPERTS]
        raw_noise = logits_all[:, NUM_EXPERTS:]
        stddev = jax.nn.softplus(raw_noise) + 1e-10
        logits = clean + noise_ref[pl.ds(r0, ct), :] * stddev
        chunks.append((logits, clean, r0))

    stats = jnp.zeros((1, 2 * NUM_EXPERTS), jnp.float32)
    z_partial = jnp.zeros((1, 1), jnp.float32)
    for logits, clean, r0 in chunks:
        s, z = _route_chunk(logits, clean, gates_ref, r0, ct)
        stats = stats + s
        z_partial = z_partial + z

    @pl.when(i == 0)
    def _():
        acc_ref[...] = jnp.zeros_like(acc_ref)

    acc_ref[0:1, :] += stats
    acc_ref[1:2, 0:1] += z_partial

    @pl.when(i == nblocks - 1)
    def _():
        imp = acc_ref[0:1, :NUM_EXPERTS]
        loads = acc_ref[0:1, NUM_EXPERTS:]
        zsum = acc_ref[1:2, 0:1]
        lb = AUX_COEF * (NUM_EXPERTS * jnp.sum(imp * loads) / float(TOKENS * TOKENS))
        zl = Z_COEF * zsum[0, 0] / float(TOKENS)
        loss_ref[...] = jnp.reshape(lb + zl, (1, 1))


def kernel(x, w_gate, w_noise):
    noise = jnp.asarray(_noise_const())
    nblocks = TOKENS // BLOCK_T

    gates, loss = pl.pallas_call(
        functools.partial(_router_kernel, nblocks=nblocks),
        grid=(nblocks,),
        in_specs=[
            pl.BlockSpec((BLOCK_T, HIDDEN), lambda i: (i, 0)),
            pl.BlockSpec((NUM_EXPERTS, HIDDEN), lambda i: (0, 0)),
            pl.BlockSpec((NUM_EXPERTS, HIDDEN), lambda i: (0, 0)),
            pl.BlockSpec((BLOCK_T, NUM_EXPERTS), lambda i: (i, 0)),
        ],
        out_specs=[
            pl.BlockSpec((BLOCK_T, NUM_EXPERTS), lambda i: (i, 0)),
            pl.BlockSpec((1, 1), lambda i: (0, 0)),
        ],
        out_shape=[
            jax.ShapeDtypeStruct((TOKENS, NUM_EXPERTS), jnp.float32),
            jax.ShapeDtypeStruct((1, 1), jnp.float32),
        ],
        scratch_shapes=[
            pltpu.VMEM((8, 2 * NUM_EXPERTS), jnp.float32),
            pltpu.VMEM((HIDDEN, 2 * NUM_EXPERTS), jnp.float32),
        ],
        compiler_params=pltpu.CompilerParams(
            dimension_semantics=("arbitrary",),
        ),
    )(x, w_gate, w_noise, noise)
    return gates, jnp.reshape(loss, ())
